# baseline (speedup 1.0000x reference)
import jax
import jax.numpy as jnp
from jax import lax
from jax.experimental import pallas as pl
from jax.experimental.pallas import tpu as pltpu
from jax.experimental.pallas import tpu_sc as plsc

EPS = 1e-5
MAX_NORM = 1.0 - 1e-3
D = 128
NC, NS = 2, 16
NW = NC * NS
N_ENT = 10000
N_ITM = 6000
N_USR = 4000

E1 = 327680
R1 = E1 // 128
K1 = R1 // NW
G1, CH1 = 10, 8
E2 = 425984
R2 = E2 // 128
K2 = R2 // NW
G2, CH2 = 13, 8
NSEG = 10112
TRASH = 10000
F = NSEG // NS


def _mk_mesh():
    return plsc.VectorSubcoreMesh(core_axis_name="c", subcore_axis_name="s",
                                  num_cores=NC, num_subcores=NS)


def _wid():
    return lax.axis_index("s") * NC + lax.axis_index("c")


KA, KB = 128, 32


def _kg_gather_body(ent, src_i, dst_i, comb_o,
                    idx_s, idx_d, b0, b1, g0, g1, w0, w1):
    c = lax.axis_index("c")
    s = lax.axis_index("s")
    row0 = jnp.where(c == 0, s * KA, NS * KA + s * KB)
    pltpu.sync_copy(src_i.at[pl.ds(row0, KB)], idx_s.at[pl.ds(0, KB)])
    pltpu.sync_copy(dst_i.at[pl.ds(row0, KB)], idx_d.at[pl.ds(0, KB)])

    @pl.when(c == 0)
    def _():
        pltpu.sync_copy(src_i.at[pl.ds(row0 + KB, KA - KB)],
                        idx_s.at[pl.ds(KB, KA - KB)])
        pltpu.sync_copy(dst_i.at[pl.ds(row0 + KB, KA - KB)],
                        idx_d.at[pl.ds(KB, KA - KB)])

    base = row0 * 256
    H = jnp.where(c == 0, KA // 2, KB // 2)

    pltpu.make_async_copy(ent.at[idx_s.at[0]], b0.at[pl.ds(0, 128)], g0).start()
    pltpu.make_async_copy(ent.at[idx_d.at[0]], b0.at[pl.ds(128, 128)], g0).start()

    def grp(g, carry):
        j0 = 2 * g
        j1 = j0 + 1
        off0 = base + j0 * 256
        off1 = off0 + 256

        @pl.when(g > 0)
        def _():
            pltpu.make_async_copy(b1, comb_o.at[pl.ds(off1, 256)], w1).wait()

        pltpu.make_async_copy(ent.at[idx_s.at[j1]], b1.at[pl.ds(0, 128)], g1).start()
        pltpu.make_async_copy(ent.at[idx_d.at[j1]], b1.at[pl.ds(128, 128)], g1).start()

        pltpu.make_async_copy(ent.at[idx_s.at[j0]], b0.at[pl.ds(0, 128)], g0).wait()
        pltpu.make_async_copy(ent.at[idx_d.at[j0]], b0.at[pl.ds(128, 128)], g0).wait()
        pltpu.make_async_copy(b0, comb_o.at[pl.ds(off0, 256)], w0).start()

        @pl.when(g < H - 1)
        def _():
            pltpu.make_async_copy(b0, comb_o.at[pl.ds(off0, 256)], w0).wait()
            pltpu.make_async_copy(ent.at[idx_s.at[j0 + 2]], b0.at[pl.ds(0, 128)], g0).start()
            pltpu.make_async_copy(ent.at[idx_d.at[j0 + 2]], b0.at[pl.ds(128, 128)], g0).start()

        pltpu.make_async_copy(ent.at[idx_s.at[j1]], b1.at[pl.ds(0, 128)], g1).wait()
        pltpu.make_async_copy(ent.at[idx_d.at[j1]], b1.at[pl.ds(128, 128)], g1).wait()
        pltpu.make_async_copy(b1, comb_o.at[pl.ds(off1, 256)], w1).start()
        return carry

    lax.fori_loop(0, H, grp, 0)

    endo = base + (2 * H - 2) * 256
    pltpu.make_async_copy(b0, comb_o.at[pl.ds(endo, 256)], w0).wait()
    pltpu.make_async_copy(b1, comb_o.at[pl.ds(endo + 256, 256)], w1).wait()


def _kg_gather(ent, src_i, dst_i):
    fn = pl.kernel(
        _kg_gather_body,
        out_type=jax.ShapeDtypeStruct((2 * E1, D), jnp.float32),
        mesh=_mk_mesh(),
        name="sc_kg_gather",
        scratch_types=[
            pltpu.VMEM((KA, 128), jnp.int32),
            pltpu.VMEM((KA, 128), jnp.int32),
            pltpu.VMEM((256, D), jnp.float32),
            pltpu.VMEM((256, D), jnp.float32),
            pltpu.SemaphoreType.DMA,
            pltpu.SemaphoreType.DMA,
            pltpu.SemaphoreType.DMA,
            pltpu.SemaphoreType.DMA,
        ],
    )
    return fn(ent, src_i, dst_i)


def _kg_scatter_body(tan, seg_i, zer_s, out_s, acc, idx_v, b0, b1, r0, r1):
    c = lax.axis_index("c")
    s = lax.axis_index("s")
    w = s * NC + c
    pltpu.sync_copy(zer_s.at[pl.ds(s * F, F)], acc.at[pl.ds(s * F, F)])
    plsc.subcore_barrier()
    base = w * (K1 * 128)

    def group(g, carry):
        pltpu.sync_copy(seg_i.at[w, g], idx_v)
        goff = base + g * (CH1 * 128)
        pltpu.make_async_copy(tan.at[pl.ds(goff, 128)], b0, r0).start()

        def pair(p, carry2):
            j0 = 2 * p
            j1 = j0 + 1
            pltpu.make_async_copy(
                tan.at[pl.ds(goff + j1 * 128, 128)], b1, r1).start()
            pltpu.make_async_copy(
                tan.at[pl.ds(goff + j0 * 128, 128)], b0, r0).wait()
            pltpu.sync_copy(b0, acc.at[idx_v.at[j0]], add=True)

            @pl.when(p < CH1 // 2 - 1)
            def _():
                pltpu.make_async_copy(
                    tan.at[pl.ds(goff + (j0 + 2) * 128, 128)], b0, r0).start()

            pltpu.make_async_copy(
                tan.at[pl.ds(goff + j1 * 128, 128)], b1, r1).wait()
            pltpu.sync_copy(b1, acc.at[idx_v.at[j1]], add=True)
            return carry2

        lax.fori_loop(0, CH1 // 2, pair, 0)
        return carry

    lax.fori_loop(0, G1, group, 0)
    plsc.subcore_barrier()
    pltpu.sync_copy(acc.at[pl.ds(s * F, F)], out_s.at[c].at[pl.ds(s * F, F)])


def _kg_scatter(tan, seg_i, zer_s):
    fn = pl.kernel(
        _kg_scatter_body,
        out_type=jax.ShapeDtypeStruct((NC, NSEG, D), jnp.float32),
        mesh=_mk_mesh(),
        name="sc_kg_scatter",
        scratch_types=[
            pltpu.VMEM_SHARED((NSEG, D), jnp.float32),
            pltpu.VMEM((CH1, 128), jnp.int32),
            pltpu.VMEM((128, D), jnp.float32),
            pltpu.VMEM((128, D), jnp.float32),
            pltpu.SemaphoreType.DMA,
            pltpu.SemaphoreType.DMA,
        ],
    )
    return fn(tan, seg_i, zer_s)


def _int_body(node, src_i, dst_i, zer_s, out_s,
              acc, idx_s, idx_d, b0, g0):
    c = lax.axis_index("c")
    s = lax.axis_index("s")
    w = s * NC + c
    pltpu.sync_copy(zer_s.at[pl.ds(s * F, F)], acc.at[pl.ds(s * F, F)])
    plsc.subcore_barrier()

    def group(g, carry):
        pltpu.sync_copy(src_i.at[w, g], idx_s)
        pltpu.sync_copy(dst_i.at[w, g], idx_d)

        def body(j, carry2):
            pltpu.make_async_copy(node.at[idx_s.at[j]], b0, g0).start()
            pltpu.make_async_copy(node.at[idx_s.at[j]], b0, g0).wait()
            pltpu.sync_copy(b0, acc.at[idx_d.at[j]], add=True)
            return carry2

        lax.fori_loop(0, CH2, body, 0)
        return carry

    lax.fori_loop(0, G2, group, 0)
    plsc.subcore_barrier()
    pltpu.sync_copy(acc.at[pl.ds(s * F, F)], out_s.at[c].at[pl.ds(s * F, F)])


def _int_agg(node, src_i, dst_i, zer_s):
    fn = pl.kernel(
        _int_body,
        out_type=jax.ShapeDtypeStruct((NC, NSEG, D), jnp.float32),
        mesh=_mk_mesh(),
        name="sc_int_agg",
        scratch_types=[
            pltpu.VMEM_SHARED((NSEG, D), jnp.float32),
            pltpu.VMEM((CH2, 128), jnp.int32),
            pltpu.VMEM((CH2, 128), jnp.int32),
            pltpu.VMEM((128, D), jnp.float32),
            pltpu.SemaphoreType.DMA,
        ],
    )
    return fn(node, src_i, dst_i, zer_s)


def _cnt_body(seg_i, dst_i, ones_h, zer_s, out_c1, out_c2,
              cnt, idx1, idx2, ones_v, sc):
    c = lax.axis_index("c")
    s = lax.axis_index("s")
    w = s * NC + c
    pltpu.sync_copy(zer_s.at[pl.ds(s * F, F)], cnt.at[pl.ds(s * F, F)])
    pltpu.sync_copy(ones_h, ones_v)
    plsc.subcore_barrier()

    def group1(g, carry):
        pltpu.sync_copy(seg_i.at[w, g], idx1)

        def fire(j, carry2):
            pltpu.make_async_copy(ones_v, cnt.at[idx1.at[j]], sc).start(
                add=True)
            return carry2

        lax.fori_loop(0, CH1, fire, 0)

        def drain(j, carry2):
            pltpu.make_async_copy(ones_v, cnt.at[idx1.at[j]], sc).wait()
            return carry2

        lax.fori_loop(0, CH1, drain, 0)
        return carry

    lax.fori_loop(0, G1, group1, 0)
    plsc.subcore_barrier()
    pltpu.sync_copy(cnt.at[pl.ds(s * F, F)], out_c1.at[c].at[pl.ds(s * F, F)])
    plsc.subcore_barrier()
    pltpu.sync_copy(zer_s.at[pl.ds(s * F, F)], cnt.at[pl.ds(s * F, F)])
    plsc.subcore_barrier()

    def group2(g, carry):
        pltpu.sync_copy(dst_i.at[w, g], idx2)

        def fire(j, carry2):
            pltpu.make_async_copy(ones_v, cnt.at[idx2.at[j]], sc).start(
                add=True)
            return carry2

        lax.fori_loop(0, CH2, fire, 0)

        def drain(j, carry2):
            pltpu.make_async_copy(ones_v, cnt.at[idx2.at[j]], sc).wait()
            return carry2

        lax.fori_loop(0, CH2, drain, 0)
        return carry

    lax.fori_loop(0, G2, group2, 0)
    plsc.subcore_barrier()
    pltpu.sync_copy(cnt.at[pl.ds(s * F, F)], out_c2.at[c].at[pl.ds(s * F, F)])


def _counts(seg_i, dst_i, ones_h, zer_s):
    fn = pl.kernel(
        _cnt_body,
        out_type=(jax.ShapeDtypeStruct((NC, NSEG, D), jnp.float32),
                  jax.ShapeDtypeStruct((NC, NSEG, D), jnp.float32)),
        mesh=_mk_mesh(),
        name="sc_counts",
        scratch_types=[
            pltpu.VMEM_SHARED((NSEG, D), jnp.float32),
            pltpu.VMEM((CH1, 128), jnp.int32),
            pltpu.VMEM((CH2, 128), jnp.int32),
            pltpu.VMEM((128, D), jnp.float32),
            pltpu.SemaphoreType.DMA,
        ],
    )
    return fn(seg_i, dst_i, ones_h, zer_s)


def _sq(x):
    return jnp.sum(x * x, axis=-1, keepdims=True)


BE = 1024


def _edge_body(comb_ref, et_ref, rt_ref, rtsq_ref, out_ref):
    u = comb_ref[:, 0].reshape(BE, D)
    p = comb_ref[:, 1].reshape(BE, D)
    et = et_ref[...]
    onehot = jnp.where(
        et + 2 == lax.broadcasted_iota(jnp.int32, (BE, 16), 1), 1.0, 0.0)
    rel = jnp.dot(onehot, rt_ref[...], preferred_element_type=jnp.float32)
    rr0 = jnp.dot(onehot, rtsq_ref[...], preferred_element_type=jnp.float32)

    uu0 = _sq(u)
    pp0 = _sq(p)
    up0 = jnp.sum(u * p, axis=-1, keepdims=True)
    ur0 = jnp.sum(u * rel, axis=-1, keepdims=True)
    pr0 = jnp.sum(p * rel, axis=-1, keepdims=True)
    S = jnp.concatenate([uu0, pp0, rr0, up0, ur0, pr0, uu0, uu0], axis=1)
    T = S.T
    uu = T[0:1]
    pp = T[1:2]
    rr = T[2:3]
    up = T[3:4]
    ur = T[4:5]
    pr = T[5:6]

    def qf(cu, cp, cr):
        return jnp.maximum(
            cu * cu * uu + cp * cp * pp + cr * cr * rr
            + 2.0 * (cu * cp * up + cu * cr * ur + cp * cr * pr), 0.0)

    n0 = jnp.maximum(jnp.sqrt(uu + 1e-15), EPS)
    sb0 = jnp.tanh(n0) / n0
    nb = jnp.sqrt(uu * sb0 * sb0 + 1e-15)
    fb = jnp.where(nb > MAX_NORM, MAX_NORM / nb, 1.0)
    sb = sb0 * fb
    bb = uu * sb * sb
    mb = jnp.maximum(1.0 - bb, EPS)

    def emap_coef(vv, uv):
        nv = jnp.maximum(jnp.sqrt(vv + 1e-15), EPS)
        sv = jnp.tanh(nv / mb) / nv
        y2 = vv * sv * sv
        xy = uv * sb * sv
        num_a = 1.0 + 2.0 * xy + y2
        num_b = 1.0 - bb
        rden = 1.0 / jnp.maximum(1.0 + 2.0 * xy + bb * y2, 1e-15)
        cb = num_a * rden * sb
        cv = num_b * rden * sv
        s2 = jnp.maximum(cb * cb * uu + 2.0 * cb * cv * uv + cv * cv * vv,
                         0.0)
        na = jnp.sqrt(s2 + 1e-15)
        fa = jnp.where(na > MAX_NORM, MAX_NORM / na, 1.0)
        return cb * fa, cv * fa, s2 * fa * fa

    au, ap, a2 = emap_coef(pp, up)
    bu, br, b2 = emap_coef(rr, ur)
    ab = au * bu * uu + au * br * ur + ap * bu * up + ap * br * pr
    a3 = 1.0 + 2.0 * ab + b2
    b3 = 1.0 - a2
    rd3 = 1.0 / jnp.maximum(1.0 + 2.0 * ab + a2 * b2, 1e-15)
    mu = a3 * rd3 * au + b3 * rd3 * bu
    mp = a3 * rd3 * ap
    mr = b3 * rd3 * br
    m2p = qf(mu, mp, mr)
    nm = jnp.sqrt(m2p + 1e-15)
    fm = jnp.where(nm > MAX_NORM, MAX_NORM / nm, 1.0)
    mu = mu * fm
    mp = mp * fm
    mr = mr * fm
    m2 = m2p * fm * fm
    bm = sb * (mu * uu + mp * up + mr * ur)
    a4 = 1.0 - 2.0 * bm + m2
    b4 = 1.0 - bb
    rd4 = 1.0 / jnp.maximum(1.0 - 2.0 * bm + bb * m2, 1e-15)
    su = -a4 * rd4 * sb + b4 * rd4 * mu
    sp = b4 * rd4 * mp
    sr = b4 * rd4 * mr
    s2s = qf(su, sp, sr)
    ns = jnp.clip(jnp.sqrt(s2s + 1e-15), EPS, 1.0 - 1e-5)
    atanh = 0.5 * jnp.log((1.0 + ns) / (1.0 - ns))
    scal = mb * atanh / ns
    cu = scal * su
    cp = scal * sp
    cr = scal * sr

    C = jnp.concatenate([cu, cp, cr, cu, cu, cu, cu, cu], axis=0)
    Ct = C.T
    out_ref[...] = (u * Ct[:, 0:1] + p * Ct[:, 1:2] + rel * Ct[:, 2:3])


def _edge_transform(comb4, et2, reltab, rtsq):
    grid = (E1 // BE,)
    nch = BE // 128
    return pl.pallas_call(
        _edge_body,
        grid=grid,
        in_specs=[
            pl.BlockSpec((nch, 2, 128, D), lambda i: (i, 0, 0, 0)),
            pl.BlockSpec((BE, 1), lambda i: (i, 0)),
            pl.BlockSpec((16, D), lambda i: (0, 0)),
            pl.BlockSpec((16, 1), lambda i: (0, 0)),
        ],
        out_specs=pl.BlockSpec((BE, D), lambda i: (i, 0)),
        out_shape=jax.ShapeDtypeStruct((E1, D), jnp.float32),
    )(comb4, et2, reltab, rtsq)


RF = 1000


def _fuse_body(e_ref, cf_ref, w1_ref, w2_ref, out_ref):
    e = e_ref[...]
    cf = cf_ref[...]
    g = jax.nn.sigmoid(
        jnp.dot(e, w1_ref[...], preferred_element_type=jnp.float32)
        + jnp.dot(cf, w2_ref[...], preferred_element_type=jnp.float32))
    out_ref[...] = g * e + (1.0 - g) * cf


def _fusion(ent_itm, cf, w1t, w2t):
    return pl.pallas_call(
        _fuse_body,
        grid=(N_ITM // RF,),
        in_specs=[
            pl.BlockSpec((RF, D), lambda i: (i, 0)),
            pl.BlockSpec((RF, D), lambda i: (i, 0)),
            pl.BlockSpec((D, D), lambda i: (0, 0)),
            pl.BlockSpec((D, D), lambda i: (0, 0)),
        ],
        out_specs=pl.BlockSpec((RF, D), lambda i: (i, 0)),
        out_shape=jax.ShapeDtypeStruct((N_ITM, D), jnp.float32),
    )(ent_itm, cf, w1t, w2t)


RB = 2528


def _final_body(s1_ref, c1_ref, s2_ref, c2_ref, o1_ref, o2_ref):
    s1 = s1_ref[0] + s1_ref[1]
    c1 = c1_ref[0, :, 0:1] + c1_ref[1, :, 0:1]
    o1_ref[...] = s1 / jnp.maximum(c1, 1.0)
    s2 = s2_ref[0] + s2_ref[1]
    c2 = c2_ref[0, :, 0:1] + c2_ref[1, :, 0:1]
    o2_ref[...] = s2 / jnp.maximum(c2, 1.0)


def _finalize(s1, c1, s2, c2):
    return pl.pallas_call(
        _final_body,
        grid=(NSEG // RB,),
        in_specs=[
            pl.BlockSpec((NC, RB, D), lambda i: (0, i, 0)),
            pl.BlockSpec((NC, RB, D), lambda i: (0, i, 0)),
            pl.BlockSpec((NC, RB, D), lambda i: (0, i, 0)),
            pl.BlockSpec((NC, RB, D), lambda i: (0, i, 0)),
        ],
        out_specs=(pl.BlockSpec((RB, D), lambda i: (i, 0)),
                   pl.BlockSpec((RB, D), lambda i: (i, 0))),
        out_shape=(jax.ShapeDtypeStruct((NSEG, D), jnp.float32),
                   jax.ShapeDtypeStruct((NSEG, D), jnp.float32)),
    )(s1, c1, s2, c2)


@jax.jit
def kernel(entity_embed, user_embed, relation_table, item_cf_embed, W1, W2,
           kg_src, kg_dst, edge_type, item_idx, user_idx):
    e_kg = kg_src.shape[0]
    e_int = item_idx.shape[0]

    pad1 = E1 - e_kg
    src_i = jnp.concatenate([kg_src, jnp.zeros((pad1,), jnp.int32)])
    dst_i = jnp.concatenate([kg_dst, jnp.zeros((pad1,), jnp.int32)])
    seg_i = jnp.concatenate([kg_src, jnp.full((pad1,), TRASH, jnp.int32)])
    et2 = jnp.concatenate([edge_type, jnp.zeros((pad1,), jnp.int32)])
    src_i = src_i.reshape(R1, 128)
    dst_i = dst_i.reshape(R1, 128)
    seg_i = seg_i.reshape(NW, G1, CH1, 128)
    et2 = et2.reshape(E1, 1)

    pad2 = E2 - 2 * e_int
    src2 = jnp.concatenate([item_idx, user_idx + N_ITM,
                            jnp.full((pad2,), TRASH, jnp.int32)])
    dst2 = jnp.concatenate([user_idx + N_ITM, item_idx,
                            jnp.full((pad2,), TRASH, jnp.int32)])
    src2 = src2.reshape(NW, G2, CH2, 128)
    dst2 = dst2.reshape(NW, G2, CH2, 128)

    reltab = jnp.concatenate(
        [relation_table, jnp.zeros((16 - relation_table.shape[0], D),
                                   jnp.float32)])
    rtsq = jnp.sum(reltab * reltab, axis=1, keepdims=True)
    ones_h = jnp.ones((128, D), jnp.float32)
    zer_s = jnp.zeros((NSEG, D), jnp.float32)

    comb = _kg_gather(entity_embed, src_i, dst_i)
    comb4 = comb.reshape(E1 // 128, 2, 128, D)

    c1, c2 = _counts(seg_i, dst2, ones_h, zer_s)

    fus = _fusion(entity_embed[:N_ITM], item_cf_embed, W1.T, W2.T)
    node = jnp.concatenate([fus, user_embed,
                            jnp.zeros((NSEG - N_ITM - N_USR, D),
                                      jnp.float32)])

    tan = _edge_transform(comb4, et2, reltab, rtsq)

    s1 = _kg_scatter(tan, seg_i, zer_s)

    s2 = _int_agg(node, src2, dst2, zer_s)

    o1, o2 = _finalize(s1, c1, s2, c2)

    out = o1[:N_ENT]
    u = o2[N_ITM:N_ITM + N_USR]
    i_cf = o2[:N_ITM]
    return (out, u, i_cf)


# speedup vs baseline: 1.3197x; 1.3197x over previous
import jax
import jax.numpy as jnp
from jax import lax
from jax.experimental import pallas as pl
from jax.experimental.pallas import tpu as pltpu
from jax.experimental.pallas import tpu_sc as plsc

EPS = 1e-5
MAX_NORM = 1.0 - 1e-3
D = 128
NC, NS = 2, 16
NW = NC * NS
N_ENT = 10000
N_ITM = 6000
N_USR = 4000

E1 = 327680
R1 = E1 // 128
K1 = R1 // NW
G1, CH1 = 10, 8
E2 = 425984
R2 = E2 // 128
K2 = R2 // NW
G2, CH2 = 13, 8
NSEG = 10112
TRASH = 10000
F = NSEG // NS


def _mk_mesh():
    return plsc.VectorSubcoreMesh(core_axis_name="c", subcore_axis_name="s",
                                  num_cores=NC, num_subcores=NS)


def _wid():
    return lax.axis_index("s") * NC + lax.axis_index("c")


KA = R1 // NS


def _kg_gather_body(ent, src_i, dst_i, comb_o,
                    idx_s, idx_d, b0, b1, g0, g1, w0, w1):
    c = lax.axis_index("c")
    s = lax.axis_index("s")

    @pl.when(c == 0)
    def _():
        row0 = s * KA
        pltpu.sync_copy(src_i.at[pl.ds(row0, KA)], idx_s)
        pltpu.sync_copy(dst_i.at[pl.ds(row0, KA)], idx_d)
        base = row0 * 256
        H = KA // 2

        pltpu.make_async_copy(ent.at[idx_s.at[0]], b0.at[pl.ds(0, 128)], g0).start()
        pltpu.make_async_copy(ent.at[idx_d.at[0]], b0.at[pl.ds(128, 128)], g0).start()

        def grp(g, carry):
            j0 = 2 * g
            j1 = j0 + 1
            off0 = base + j0 * 256
            off1 = off0 + 256

            @pl.when(g > 0)
            def _():
                pltpu.make_async_copy(b1, comb_o.at[pl.ds(off1, 256)], w1).wait()

            pltpu.make_async_copy(ent.at[idx_s.at[j1]], b1.at[pl.ds(0, 128)], g1).start()
            pltpu.make_async_copy(ent.at[idx_d.at[j1]], b1.at[pl.ds(128, 128)], g1).start()

            pltpu.make_async_copy(ent.at[idx_s.at[j0]], b0.at[pl.ds(0, 128)], g0).wait()
            pltpu.make_async_copy(ent.at[idx_d.at[j0]], b0.at[pl.ds(128, 128)], g0).wait()
            pltpu.make_async_copy(b0, comb_o.at[pl.ds(off0, 256)], w0).start()

            @pl.when(g < H - 1)
            def _():
                pltpu.make_async_copy(b0, comb_o.at[pl.ds(off0, 256)], w0).wait()
                pltpu.make_async_copy(ent.at[idx_s.at[j0 + 2]], b0.at[pl.ds(0, 128)], g0).start()
                pltpu.make_async_copy(ent.at[idx_d.at[j0 + 2]], b0.at[pl.ds(128, 128)], g0).start()

            pltpu.make_async_copy(ent.at[idx_s.at[j1]], b1.at[pl.ds(0, 128)], g1).wait()
            pltpu.make_async_copy(ent.at[idx_d.at[j1]], b1.at[pl.ds(128, 128)], g1).wait()
            pltpu.make_async_copy(b1, comb_o.at[pl.ds(off1, 256)], w1).start()
            return carry

        lax.fori_loop(0, H, grp, 0)

        endo = base + (KA - 2) * 256
        pltpu.make_async_copy(b0, comb_o.at[pl.ds(endo, 256)], w0).wait()
        pltpu.make_async_copy(b1, comb_o.at[pl.ds(endo + 256, 256)], w1).wait()


def _kg_gather(ent, src_i, dst_i):
    fn = pl.kernel(
        _kg_gather_body,
        out_type=jax.ShapeDtypeStruct((2 * E1, D), jnp.float32),
        mesh=_mk_mesh(),
        name="sc_kg_gather",
        scratch_types=[
            pltpu.VMEM((KA, 128), jnp.int32),
            pltpu.VMEM((KA, 128), jnp.int32),
            pltpu.VMEM((256, D), jnp.float32),
            pltpu.VMEM((256, D), jnp.float32),
            pltpu.SemaphoreType.DMA,
            pltpu.SemaphoreType.DMA,
            pltpu.SemaphoreType.DMA,
            pltpu.SemaphoreType.DMA,
        ],
    )
    return fn(ent, src_i, dst_i)


def _kg_scatter_body(tan, seg_i, zer_s, out_s, acc, idx_v, b0, b1, r0, r1):
    c = lax.axis_index("c")
    s = lax.axis_index("s")
    w = s * NC + c
    pltpu.sync_copy(zer_s.at[pl.ds(s * F, F)], acc.at[pl.ds(s * F, F)])
    plsc.subcore_barrier()
    base = w * (K1 * 128)

    def group(g, carry):
        pltpu.sync_copy(seg_i.at[w, g], idx_v)
        goff = base + g * (CH1 * 128)
        pltpu.make_async_copy(tan.at[pl.ds(goff, 128)], b0, r0).start()

        def pair(p, carry2):
            j0 = 2 * p
            j1 = j0 + 1
            pltpu.make_async_copy(
                tan.at[pl.ds(goff + j1 * 128, 128)], b1, r1).start()
            pltpu.make_async_copy(
                tan.at[pl.ds(goff + j0 * 128, 128)], b0, r0).wait()
            pltpu.sync_copy(b0, acc.at[idx_v.at[j0]], add=True)

            @pl.when(p < CH1 // 2 - 1)
            def _():
                pltpu.make_async_copy(
                    tan.at[pl.ds(goff + (j0 + 2) * 128, 128)], b0, r0).start()

            pltpu.make_async_copy(
                tan.at[pl.ds(goff + j1 * 128, 128)], b1, r1).wait()
            pltpu.sync_copy(b1, acc.at[idx_v.at[j1]], add=True)
            return carry2

        lax.fori_loop(0, CH1 // 2, pair, 0)
        return carry

    lax.fori_loop(0, G1, group, 0)
    plsc.subcore_barrier()
    pltpu.sync_copy(acc.at[pl.ds(s * F, F)], out_s.at[c].at[pl.ds(s * F, F)])


def _kg_scatter(tan, seg_i, zer_s):
    fn = pl.kernel(
        _kg_scatter_body,
        out_type=jax.ShapeDtypeStruct((NC, NSEG, D), jnp.float32),
        mesh=_mk_mesh(),
        name="sc_kg_scatter",
        scratch_types=[
            pltpu.VMEM_SHARED((NSEG, D), jnp.float32),
            pltpu.VMEM((CH1, 128), jnp.int32),
            pltpu.VMEM((128, D), jnp.float32),
            pltpu.VMEM((128, D), jnp.float32),
            pltpu.SemaphoreType.DMA,
            pltpu.SemaphoreType.DMA,
        ],
    )
    return fn(tan, seg_i, zer_s)


def _int_body(node, src_i, dst_i, zer_s, out_s,
              acc, idx_s, idx_d, b0, g0):
    c = lax.axis_index("c")
    s = lax.axis_index("s")
    w = s * NC + c
    pltpu.sync_copy(zer_s.at[pl.ds(s * F, F)], acc.at[pl.ds(s * F, F)])
    plsc.subcore_barrier()

    def group(g, carry):
        pltpu.sync_copy(src_i.at[w, g], idx_s)
        pltpu.sync_copy(dst_i.at[w, g], idx_d)

        def body(j, carry2):
            pltpu.make_async_copy(node.at[idx_s.at[j]], b0, g0).start()
            pltpu.make_async_copy(node.at[idx_s.at[j]], b0, g0).wait()
            pltpu.sync_copy(b0, acc.at[idx_d.at[j]], add=True)
            return carry2

        lax.fori_loop(0, CH2, body, 0)
        return carry

    lax.fori_loop(0, G2, group, 0)
    plsc.subcore_barrier()
    pltpu.sync_copy(acc.at[pl.ds(s * F, F)], out_s.at[c].at[pl.ds(s * F, F)])


def _int_agg(node, src_i, dst_i, zer_s):
    fn = pl.kernel(
        _int_body,
        out_type=jax.ShapeDtypeStruct((NC, NSEG, D), jnp.float32),
        mesh=_mk_mesh(),
        name="sc_int_agg",
        scratch_types=[
            pltpu.VMEM_SHARED((NSEG, D), jnp.float32),
            pltpu.VMEM((CH2, 128), jnp.int32),
            pltpu.VMEM((CH2, 128), jnp.int32),
            pltpu.VMEM((128, D), jnp.float32),
            pltpu.SemaphoreType.DMA,
        ],
    )
    return fn(node, src_i, dst_i, zer_s)


def _cnt_body(seg_i, dst_i, ones_h, zer_s, out_c1, out_c2,
              cnt, idx1, idx2, ones_v, sc):
    c = lax.axis_index("c")
    s = lax.axis_index("s")
    w = s * NC + c
    pltpu.sync_copy(zer_s.at[pl.ds(s * F, F)], cnt.at[pl.ds(s * F, F)])
    pltpu.sync_copy(ones_h, ones_v)
    plsc.subcore_barrier()

    def group1(g, carry):
        pltpu.sync_copy(seg_i.at[w, g], idx1)

        def fire(j, carry2):
            pltpu.make_async_copy(ones_v, cnt.at[idx1.at[j]], sc).start(
                add=True)
            return carry2

        lax.fori_loop(0, CH1, fire, 0)

        def drain(j, carry2):
            pltpu.make_async_copy(ones_v, cnt.at[idx1.at[j]], sc).wait()
            return carry2

        lax.fori_loop(0, CH1, drain, 0)
        return carry

    lax.fori_loop(0, G1, group1, 0)
    plsc.subcore_barrier()
    pltpu.sync_copy(cnt.at[pl.ds(s * F, F)], out_c1.at[c].at[pl.ds(s * F, F)])
    plsc.subcore_barrier()
    pltpu.sync_copy(zer_s.at[pl.ds(s * F, F)], cnt.at[pl.ds(s * F, F)])
    plsc.subcore_barrier()

    def group2(g, carry):
        pltpu.sync_copy(dst_i.at[w, g], idx2)

        def fire(j, carry2):
            pltpu.make_async_copy(ones_v, cnt.at[idx2.at[j]], sc).start(
                add=True)
            return carry2

        lax.fori_loop(0, CH2, fire, 0)

        def drain(j, carry2):
            pltpu.make_async_copy(ones_v, cnt.at[idx2.at[j]], sc).wait()
            return carry2

        lax.fori_loop(0, CH2, drain, 0)
        return carry

    lax.fori_loop(0, G2, group2, 0)
    plsc.subcore_barrier()
    pltpu.sync_copy(cnt.at[pl.ds(s * F, F)], out_c2.at[c].at[pl.ds(s * F, F)])


def _counts(seg_i, dst_i, ones_h, zer_s):
    fn = pl.kernel(
        _cnt_body,
        out_type=(jax.ShapeDtypeStruct((NC, NSEG, D), jnp.float32),
                  jax.ShapeDtypeStruct((NC, NSEG, D), jnp.float32)),
        mesh=_mk_mesh(),
        name="sc_counts",
        scratch_types=[
            pltpu.VMEM_SHARED((NSEG, D), jnp.float32),
            pltpu.VMEM((CH1, 128), jnp.int32),
            pltpu.VMEM((CH2, 128), jnp.int32),
            pltpu.VMEM((128, D), jnp.float32),
            pltpu.SemaphoreType.DMA,
        ],
    )
    return fn(seg_i, dst_i, ones_h, zer_s)


def _sq(x):
    return jnp.sum(x * x, axis=-1, keepdims=True)


BE = 1024


def _edge_body(comb_ref, et_ref, rt_ref, rtsq_ref, out_ref):
    u = comb_ref[:, 0].reshape(BE, D)
    p = comb_ref[:, 1].reshape(BE, D)
    et = et_ref[...]
    onehot = jnp.where(
        et + 2 == lax.broadcasted_iota(jnp.int32, (BE, 16), 1), 1.0, 0.0)
    rel = jnp.dot(onehot, rt_ref[...], preferred_element_type=jnp.float32)
    rr0 = jnp.dot(onehot, rtsq_ref[...], preferred_element_type=jnp.float32)

    uu0 = _sq(u)
    pp0 = _sq(p)
    up0 = jnp.sum(u * p, axis=-1, keepdims=True)
    ur0 = jnp.sum(u * rel, axis=-1, keepdims=True)
    pr0 = jnp.sum(p * rel, axis=-1, keepdims=True)
    S = jnp.concatenate([uu0, pp0, rr0, up0, ur0, pr0, uu0, uu0], axis=1)
    T = S.T
    uu = T[0:1]
    pp = T[1:2]
    rr = T[2:3]
    up = T[3:4]
    ur = T[4:5]
    pr = T[5:6]

    def qf(cu, cp, cr):
        return jnp.maximum(
            cu * cu * uu + cp * cp * pp + cr * cr * rr
            + 2.0 * (cu * cp * up + cu * cr * ur + cp * cr * pr), 0.0)

    n0 = jnp.maximum(jnp.sqrt(uu + 1e-15), EPS)
    sb0 = jnp.tanh(n0) / n0
    nb = jnp.sqrt(uu * sb0 * sb0 + 1e-15)
    fb = jnp.where(nb > MAX_NORM, MAX_NORM / nb, 1.0)
    sb = sb0 * fb
    bb = uu * sb * sb
    mb = jnp.maximum(1.0 - bb, EPS)

    def emap_coef(vv, uv):
        nv = jnp.maximum(jnp.sqrt(vv + 1e-15), EPS)
        sv = jnp.tanh(nv / mb) / nv
        y2 = vv * sv * sv
        xy = uv * sb * sv
        num_a = 1.0 + 2.0 * xy + y2
        num_b = 1.0 - bb
        rden = 1.0 / jnp.maximum(1.0 + 2.0 * xy + bb * y2, 1e-15)
        cb = num_a * rden * sb
        cv = num_b * rden * sv
        s2 = jnp.maximum(cb * cb * uu + 2.0 * cb * cv * uv + cv * cv * vv,
                         0.0)
        na = jnp.sqrt(s2 + 1e-15)
        fa = jnp.where(na > MAX_NORM, MAX_NORM / na, 1.0)
        return cb * fa, cv * fa, s2 * fa * fa

    au, ap, a2 = emap_coef(pp, up)
    bu, br, b2 = emap_coef(rr, ur)
    ab = au * bu * uu + au * br * ur + ap * bu * up + ap * br * pr
    a3 = 1.0 + 2.0 * ab + b2
    b3 = 1.0 - a2
    rd3 = 1.0 / jnp.maximum(1.0 + 2.0 * ab + a2 * b2, 1e-15)
    mu = a3 * rd3 * au + b3 * rd3 * bu
    mp = a3 * rd3 * ap
    mr = b3 * rd3 * br
    m2p = qf(mu, mp, mr)
    nm = jnp.sqrt(m2p + 1e-15)
    fm = jnp.where(nm > MAX_NORM, MAX_NORM / nm, 1.0)
    mu = mu * fm
    mp = mp * fm
    mr = mr * fm
    m2 = m2p * fm * fm
    bm = sb * (mu * uu + mp * up + mr * ur)
    a4 = 1.0 - 2.0 * bm + m2
    b4 = 1.0 - bb
    rd4 = 1.0 / jnp.maximum(1.0 - 2.0 * bm + bb * m2, 1e-15)
    su = -a4 * rd4 * sb + b4 * rd4 * mu
    sp = b4 * rd4 * mp
    sr = b4 * rd4 * mr
    s2s = qf(su, sp, sr)
    ns = jnp.clip(jnp.sqrt(s2s + 1e-15), EPS, 1.0 - 1e-5)
    atanh = 0.5 * jnp.log((1.0 + ns) / (1.0 - ns))
    scal = mb * atanh / ns
    cu = scal * su
    cp = scal * sp
    cr = scal * sr

    C = jnp.concatenate([cu, cp, cr, cu, cu, cu, cu, cu], axis=0)
    Ct = C.T
    out_ref[...] = (u * Ct[:, 0:1] + p * Ct[:, 1:2] + rel * Ct[:, 2:3])


def _edge_transform(comb4, et2, reltab, rtsq):
    grid = (E1 // BE,)
    nch = BE // 128
    return pl.pallas_call(
        _edge_body,
        grid=grid,
        in_specs=[
            pl.BlockSpec((nch, 2, 128, D), lambda i: (i, 0, 0, 0)),
            pl.BlockSpec((BE, 1), lambda i: (i, 0)),
            pl.BlockSpec((16, D), lambda i: (0, 0)),
            pl.BlockSpec((16, 1), lambda i: (0, 0)),
        ],
        out_specs=pl.BlockSpec((BE, D), lambda i: (i, 0)),
        out_shape=jax.ShapeDtypeStruct((E1, D), jnp.float32),
    )(comb4, et2, reltab, rtsq)


RF = 1000


def _fuse_body(e_ref, cf_ref, w1_ref, w2_ref, out_ref):
    e = e_ref[...]
    cf = cf_ref[...]
    g = jax.nn.sigmoid(
        jnp.dot(e, w1_ref[...], preferred_element_type=jnp.float32)
        + jnp.dot(cf, w2_ref[...], preferred_element_type=jnp.float32))
    out_ref[...] = g * e + (1.0 - g) * cf


def _fusion(ent_itm, cf, w1t, w2t):
    return pl.pallas_call(
        _fuse_body,
        grid=(N_ITM // RF,),
        in_specs=[
            pl.BlockSpec((RF, D), lambda i: (i, 0)),
            pl.BlockSpec((RF, D), lambda i: (i, 0)),
            pl.BlockSpec((D, D), lambda i: (0, 0)),
            pl.BlockSpec((D, D), lambda i: (0, 0)),
        ],
        out_specs=pl.BlockSpec((RF, D), lambda i: (i, 0)),
        out_shape=jax.ShapeDtypeStruct((N_ITM, D), jnp.float32),
    )(ent_itm, cf, w1t, w2t)


RB = 2528


def _final_body(s1_ref, c1_ref, s2_ref, c2_ref, o1_ref, o2_ref):
    s1 = s1_ref[0] + s1_ref[1]
    c1 = c1_ref[0, :, 0:1] + c1_ref[1, :, 0:1]
    o1_ref[...] = s1 / jnp.maximum(c1, 1.0)
    s2 = s2_ref[0] + s2_ref[1]
    c2 = c2_ref[0, :, 0:1] + c2_ref[1, :, 0:1]
    o2_ref[...] = s2 / jnp.maximum(c2, 1.0)


def _finalize(s1, c1, s2, c2):
    return pl.pallas_call(
        _final_body,
        grid=(NSEG // RB,),
        in_specs=[
            pl.BlockSpec((NC, RB, D), lambda i: (0, i, 0)),
            pl.BlockSpec((NC, RB, D), lambda i: (0, i, 0)),
            pl.BlockSpec((NC, RB, D), lambda i: (0, i, 0)),
            pl.BlockSpec((NC, RB, D), lambda i: (0, i, 0)),
        ],
        out_specs=(pl.BlockSpec((RB, D), lambda i: (i, 0)),
                   pl.BlockSpec((RB, D), lambda i: (i, 0))),
        out_shape=(jax.ShapeDtypeStruct((NSEG, D), jnp.float32),
                   jax.ShapeDtypeStruct((NSEG, D), jnp.float32)),
    )(s1, c1, s2, c2)


@jax.jit
def kernel(entity_embed, user_embed, relation_table, item_cf_embed, W1, W2,
           kg_src, kg_dst, edge_type, item_idx, user_idx):
    e_kg = kg_src.shape[0]
    e_int = item_idx.shape[0]

    pad1 = E1 - e_kg
    trash1 = TRASH + jnp.arange(pad1, dtype=jnp.int32) % (NSEG - TRASH)
    src_i = jnp.concatenate([kg_src, jnp.zeros((pad1,), jnp.int32)])
    dst_i = jnp.concatenate([kg_dst, jnp.zeros((pad1,), jnp.int32)])
    seg_i = jnp.concatenate([kg_src, trash1])
    et2 = jnp.concatenate([edge_type, jnp.zeros((pad1,), jnp.int32)])
    src_i = src_i.reshape(R1, 128)
    dst_i = dst_i.reshape(R1, 128)
    seg_i = seg_i.reshape(NW, G1, CH1, 128)
    et2 = et2.reshape(E1, 1)

    pad2 = E2 - 2 * e_int
    trash2 = TRASH + jnp.arange(pad2, dtype=jnp.int32) % (NSEG - TRASH)
    src2 = jnp.concatenate([item_idx, user_idx + N_ITM, trash2])
    dst2 = jnp.concatenate([user_idx + N_ITM, item_idx, trash2])
    src2 = src2.reshape(NW, G2, CH2, 128)
    dst2 = dst2.reshape(NW, G2, CH2, 128)

    reltab = jnp.concatenate(
        [relation_table, jnp.zeros((16 - relation_table.shape[0], D),
                                   jnp.float32)])
    rtsq = jnp.sum(reltab * reltab, axis=1, keepdims=True)
    ones_h = jnp.ones((128, D), jnp.float32)
    zer_s = jnp.zeros((NSEG, D), jnp.float32)

    comb = _kg_gather(entity_embed, src_i, dst_i)
    comb4 = comb.reshape(E1 // 128, 2, 128, D)

    c1, c2 = _counts(seg_i, dst2, ones_h, zer_s)

    fus = _fusion(entity_embed[:N_ITM], item_cf_embed, W1.T, W2.T)
    node = jnp.concatenate([fus, user_embed,
                            jnp.zeros((NSEG - N_ITM - N_USR, D),
                                      jnp.float32)])

    tan = _edge_transform(comb4, et2, reltab, rtsq)

    s1 = _kg_scatter(tan, seg_i, zer_s)

    s2 = _int_agg(node, src2, dst2, zer_s)

    o1, o2 = _finalize(s1, c1, s2, c2)

    out = o1[:N_ENT]
    u = o2[N_ITM:N_ITM + N_USR]
    i_cf = o2[:N_ITM]
    return (out, u, i_cf)


# speedup vs baseline: 1.3970x; 1.0586x over previous
import jax
import jax.numpy as jnp
from jax import lax
from jax.experimental import pallas as pl
from jax.experimental.pallas import tpu as pltpu
from jax.experimental.pallas import tpu_sc as plsc

EPS = 1e-5
MAX_NORM = 1.0 - 1e-3
D = 128
NC, NS = 2, 16
NW = NC * NS
N_ENT = 10000
N_ITM = 6000
N_USR = 4000

E1 = 327680
R1 = E1 // 128
K1 = R1 // NW
G1, CH1 = 10, 8
E2 = 425984
R2 = E2 // 128
K2 = R2 // NW
G2, CH2 = 13, 8
NSEG = 10112
TRASH = 10000
F = NSEG // NS


def _mk_mesh():
    return plsc.VectorSubcoreMesh(core_axis_name="c", subcore_axis_name="s",
                                  num_cores=NC, num_subcores=NS)


def _wid():
    return lax.axis_index("s") * NC + lax.axis_index("c")


def _kg_gather_body(ent, src_i, dst_i, comb_o,
                    idx_s, idx_d, b0, b1, g0, g1, w0, w1):
    w = _wid()
    row0 = w * K1
    pltpu.sync_copy(src_i.at[pl.ds(row0, K1)], idx_s)
    pltpu.sync_copy(dst_i.at[pl.ds(row0, K1)], idx_d)
    base = row0 * 256
    H = K1 // 2

    pltpu.make_async_copy(ent.at[idx_s.at[0]], b0.at[pl.ds(0, 128)], g0).start()
    pltpu.make_async_copy(ent.at[idx_d.at[0]], b0.at[pl.ds(128, 128)], g0).start()

    def grp(g, carry):
        j0 = 2 * g
        j1 = j0 + 1
        off0 = base + j0 * 256
        off1 = off0 + 256

        @pl.when(g > 0)
        def _():
            pltpu.make_async_copy(b1, comb_o.at[pl.ds(off1, 256)], w1).wait()

        pltpu.make_async_copy(ent.at[idx_s.at[j1]], b1.at[pl.ds(0, 128)], g1).start()
        pltpu.make_async_copy(ent.at[idx_d.at[j1]], b1.at[pl.ds(128, 128)], g1).start()

        pltpu.make_async_copy(ent.at[idx_s.at[j0]], b0.at[pl.ds(0, 128)], g0).wait()
        pltpu.make_async_copy(ent.at[idx_d.at[j0]], b0.at[pl.ds(128, 128)], g0).wait()
        pltpu.make_async_copy(b0, comb_o.at[pl.ds(off0, 256)], w0).start()

        @pl.when(g < H - 1)
        def _():
            pltpu.make_async_copy(b0, comb_o.at[pl.ds(off0, 256)], w0).wait()
            pltpu.make_async_copy(ent.at[idx_s.at[j0 + 2]], b0.at[pl.ds(0, 128)], g0).start()
            pltpu.make_async_copy(ent.at[idx_d.at[j0 + 2]], b0.at[pl.ds(128, 128)], g0).start()

        pltpu.make_async_copy(ent.at[idx_s.at[j1]], b1.at[pl.ds(0, 128)], g1).wait()
        pltpu.make_async_copy(ent.at[idx_d.at[j1]], b1.at[pl.ds(128, 128)], g1).wait()
        pltpu.make_async_copy(b1, comb_o.at[pl.ds(off1, 256)], w1).start()
        return carry

    lax.fori_loop(0, H, grp, 0)

    endo = base + (K1 - 2) * 256
    pltpu.make_async_copy(b0, comb_o.at[pl.ds(endo, 256)], w0).wait()
    pltpu.make_async_copy(b1, comb_o.at[pl.ds(endo + 256, 256)], w1).wait()


def _kg_gather(ent, src_i, dst_i):
    fn = pl.kernel(
        _kg_gather_body,
        out_type=jax.ShapeDtypeStruct((2 * E1, D), jnp.float32),
        mesh=_mk_mesh(),
        name="sc_kg_gather",
        scratch_types=[
            pltpu.VMEM((K1, 128), jnp.int32),
            pltpu.VMEM((K1, 128), jnp.int32),
            pltpu.VMEM((256, D), jnp.float32),
            pltpu.VMEM((256, D), jnp.float32),
            pltpu.SemaphoreType.DMA,
            pltpu.SemaphoreType.DMA,
            pltpu.SemaphoreType.DMA,
            pltpu.SemaphoreType.DMA,
        ],
    )
    return fn(ent, src_i, dst_i)


def _kg_scatter_body(tan, seg_i, zer_s, out_s, acc, idx_v, b0, b1, r0, r1):
    c = lax.axis_index("c")
    s = lax.axis_index("s")
    w = s * NC + c
    pltpu.sync_copy(zer_s.at[pl.ds(s * F, F)], acc.at[pl.ds(s * F, F)])
    plsc.subcore_barrier()
    base = w * (K1 * 128)

    def group(g, carry):
        pltpu.sync_copy(seg_i.at[w, g], idx_v)
        goff = base + g * (CH1 * 128)
        pltpu.make_async_copy(tan.at[pl.ds(goff, 128)], b0, r0).start()

        def pair(p, carry2):
            j0 = 2 * p
            j1 = j0 + 1
            pltpu.make_async_copy(
                tan.at[pl.ds(goff + j1 * 128, 128)], b1, r1).start()
            pltpu.make_async_copy(
                tan.at[pl.ds(goff + j0 * 128, 128)], b0, r0).wait()
            pltpu.sync_copy(b0, acc.at[idx_v.at[j0]], add=True)

            @pl.when(p < CH1 // 2 - 1)
            def _():
                pltpu.make_async_copy(
                    tan.at[pl.ds(goff + (j0 + 2) * 128, 128)], b0, r0).start()

            pltpu.make_async_copy(
                tan.at[pl.ds(goff + j1 * 128, 128)], b1, r1).wait()
            pltpu.sync_copy(b1, acc.at[idx_v.at[j1]], add=True)
            return carry2

        lax.fori_loop(0, CH1 // 2, pair, 0)
        return carry

    lax.fori_loop(0, G1, group, 0)
    plsc.subcore_barrier()
    pltpu.sync_copy(acc.at[pl.ds(s * F, F)], out_s.at[c].at[pl.ds(s * F, F)])


def _kg_scatter(tan, seg_i, zer_s):
    fn = pl.kernel(
        _kg_scatter_body,
        out_type=jax.ShapeDtypeStruct((NC, NSEG, D), jnp.float32),
        mesh=_mk_mesh(),
        name="sc_kg_scatter",
        scratch_types=[
            pltpu.VMEM_SHARED((NSEG, D), jnp.float32),
            pltpu.VMEM((CH1, 128), jnp.int32),
            pltpu.VMEM((128, D), jnp.float32),
            pltpu.VMEM((128, D), jnp.float32),
            pltpu.SemaphoreType.DMA,
            pltpu.SemaphoreType.DMA,
        ],
    )
    return fn(tan, seg_i, zer_s)


def _int_body(node, src_i, dst_i, zer_s, out_s,
              acc, idx_s, idx_d, b0, g0):
    c = lax.axis_index("c")
    s = lax.axis_index("s")
    w = s * NC + c
    pltpu.sync_copy(zer_s.at[pl.ds(s * F, F)], acc.at[pl.ds(s * F, F)])
    plsc.subcore_barrier()

    def group(g, carry):
        pltpu.sync_copy(src_i.at[w, g], idx_s)
        pltpu.sync_copy(dst_i.at[w, g], idx_d)

        def body(j, carry2):
            pltpu.make_async_copy(node.at[idx_s.at[j]], b0, g0).start()
            pltpu.make_async_copy(node.at[idx_s.at[j]], b0, g0).wait()
            pltpu.sync_copy(b0, acc.at[idx_d.at[j]], add=True)
            return carry2

        lax.fori_loop(0, CH2, body, 0)
        return carry

    lax.fori_loop(0, G2, group, 0)
    plsc.subcore_barrier()
    pltpu.sync_copy(acc.at[pl.ds(s * F, F)], out_s.at[c].at[pl.ds(s * F, F)])


def _int_agg(node, src_i, dst_i, zer_s):
    fn = pl.kernel(
        _int_body,
        out_type=jax.ShapeDtypeStruct((NC, NSEG, D), jnp.float32),
        mesh=_mk_mesh(),
        name="sc_int_agg",
        scratch_types=[
            pltpu.VMEM_SHARED((NSEG, D), jnp.float32),
            pltpu.VMEM((CH2, 128), jnp.int32),
            pltpu.VMEM((CH2, 128), jnp.int32),
            pltpu.VMEM((128, D), jnp.float32),
            pltpu.SemaphoreType.DMA,
        ],
    )
    return fn(node, src_i, dst_i, zer_s)


def _cnt_body(seg_i, dst_i, ones_h, zer_s, out_c1, out_c2,
              cnt, idx1, idx2, ones_v, sc):
    c = lax.axis_index("c")
    s = lax.axis_index("s")
    w = s * NC + c
    pltpu.sync_copy(zer_s.at[pl.ds(s * F, F)], cnt.at[pl.ds(s * F, F)])
    pltpu.sync_copy(ones_h, ones_v)
    plsc.subcore_barrier()

    def group1(g, carry):
        pltpu.sync_copy(seg_i.at[w, g], idx1)

        def fire(j, carry2):
            pltpu.make_async_copy(ones_v, cnt.at[idx1.at[j]], sc).start(
                add=True)
            return carry2

        lax.fori_loop(0, CH1, fire, 0)

        def drain(j, carry2):
            pltpu.make_async_copy(ones_v, cnt.at[idx1.at[j]], sc).wait()
            return carry2

        lax.fori_loop(0, CH1, drain, 0)
        return carry

    lax.fori_loop(0, G1, group1, 0)
    plsc.subcore_barrier()
    pltpu.sync_copy(cnt.at[pl.ds(s * F, F)], out_c1.at[c].at[pl.ds(s * F, F)])
    plsc.subcore_barrier()
    pltpu.sync_copy(zer_s.at[pl.ds(s * F, F)], cnt.at[pl.ds(s * F, F)])
    plsc.subcore_barrier()

    def group2(g, carry):
        pltpu.sync_copy(dst_i.at[w, g], idx2)

        def fire(j, carry2):
            pltpu.make_async_copy(ones_v, cnt.at[idx2.at[j]], sc).start(
                add=True)
            return carry2

        lax.fori_loop(0, CH2, fire, 0)

        def drain(j, carry2):
            pltpu.make_async_copy(ones_v, cnt.at[idx2.at[j]], sc).wait()
            return carry2

        lax.fori_loop(0, CH2, drain, 0)
        return carry

    lax.fori_loop(0, G2, group2, 0)
    plsc.subcore_barrier()
    pltpu.sync_copy(cnt.at[pl.ds(s * F, F)], out_c2.at[c].at[pl.ds(s * F, F)])


def _counts(seg_i, dst_i, ones_h, zer_s):
    fn = pl.kernel(
        _cnt_body,
        out_type=(jax.ShapeDtypeStruct((NC, NSEG, D), jnp.float32),
                  jax.ShapeDtypeStruct((NC, NSEG, D), jnp.float32)),
        mesh=_mk_mesh(),
        name="sc_counts",
        scratch_types=[
            pltpu.VMEM_SHARED((NSEG, D), jnp.float32),
            pltpu.VMEM((CH1, 128), jnp.int32),
            pltpu.VMEM((CH2, 128), jnp.int32),
            pltpu.VMEM((128, D), jnp.float32),
            pltpu.SemaphoreType.DMA,
        ],
    )
    return fn(seg_i, dst_i, ones_h, zer_s)


def _sq(x):
    return jnp.sum(x * x, axis=-1, keepdims=True)


BE = 1024


def _edge_body(comb_ref, et_ref, rt_ref, rtsq_ref, out_ref):
    u = comb_ref[:, 0].reshape(BE, D)
    p = comb_ref[:, 1].reshape(BE, D)
    et = et_ref[...]
    onehot = jnp.where(
        et + 2 == lax.broadcasted_iota(jnp.int32, (BE, 16), 1), 1.0, 0.0)
    rel = jnp.dot(onehot, rt_ref[...], preferred_element_type=jnp.float32)
    rr0 = jnp.dot(onehot, rtsq_ref[...], preferred_element_type=jnp.float32)

    uu0 = _sq(u)
    pp0 = _sq(p)
    up0 = jnp.sum(u * p, axis=-1, keepdims=True)
    ur0 = jnp.sum(u * rel, axis=-1, keepdims=True)
    pr0 = jnp.sum(p * rel, axis=-1, keepdims=True)
    S = jnp.concatenate([uu0, pp0, rr0, up0, ur0, pr0, uu0, uu0], axis=1)
    T = S.T
    uu = T[0:1]
    pp = T[1:2]
    rr = T[2:3]
    up = T[3:4]
    ur = T[4:5]
    pr = T[5:6]

    def qf(cu, cp, cr):
        return jnp.maximum(
            cu * cu * uu + cp * cp * pp + cr * cr * rr
            + 2.0 * (cu * cp * up + cu * cr * ur + cp * cr * pr), 0.0)

    n0 = jnp.maximum(jnp.sqrt(uu + 1e-15), EPS)
    sb0 = jnp.tanh(n0) / n0
    nb = jnp.sqrt(uu * sb0 * sb0 + 1e-15)
    fb = jnp.where(nb > MAX_NORM, MAX_NORM / nb, 1.0)
    sb = sb0 * fb
    bb = uu * sb * sb
    mb = jnp.maximum(1.0 - bb, EPS)

    def emap_coef(vv, uv):
        nv = jnp.maximum(jnp.sqrt(vv + 1e-15), EPS)
        sv = jnp.tanh(nv / mb) / nv
        y2 = vv * sv * sv
        xy = uv * sb * sv
        num_a = 1.0 + 2.0 * xy + y2
        num_b = 1.0 - bb
        rden = 1.0 / jnp.maximum(1.0 + 2.0 * xy + bb * y2, 1e-15)
        cb = num_a * rden * sb
        cv = num_b * rden * sv
        s2 = jnp.maximum(cb * cb * uu + 2.0 * cb * cv * uv + cv * cv * vv,
                         0.0)
        na = jnp.sqrt(s2 + 1e-15)
        fa = jnp.where(na > MAX_NORM, MAX_NORM / na, 1.0)
        return cb * fa, cv * fa, s2 * fa * fa

    au, ap, a2 = emap_coef(pp, up)
    bu, br, b2 = emap_coef(rr, ur)
    ab = au * bu * uu + au * br * ur + ap * bu * up + ap * br * pr
    a3 = 1.0 + 2.0 * ab + b2
    b3 = 1.0 - a2
    rd3 = 1.0 / jnp.maximum(1.0 + 2.0 * ab + a2 * b2, 1e-15)
    mu = a3 * rd3 * au + b3 * rd3 * bu
    mp = a3 * rd3 * ap
    mr = b3 * rd3 * br
    m2p = qf(mu, mp, mr)
    nm = jnp.sqrt(m2p + 1e-15)
    fm = jnp.where(nm > MAX_NORM, MAX_NORM / nm, 1.0)
    mu = mu * fm
    mp = mp * fm
    mr = mr * fm
    m2 = m2p * fm * fm
    bm = sb * (mu * uu + mp * up + mr * ur)
    a4 = 1.0 - 2.0 * bm + m2
    b4 = 1.0 - bb
    rd4 = 1.0 / jnp.maximum(1.0 - 2.0 * bm + bb * m2, 1e-15)
    su = -a4 * rd4 * sb + b4 * rd4 * mu
    sp = b4 * rd4 * mp
    sr = b4 * rd4 * mr
    s2s = qf(su, sp, sr)
    ns = jnp.clip(jnp.sqrt(s2s + 1e-15), EPS, 1.0 - 1e-5)
    atanh = 0.5 * jnp.log((1.0 + ns) / (1.0 - ns))
    scal = mb * atanh / ns
    cu = scal * su
    cp = scal * sp
    cr = scal * sr

    C = jnp.concatenate([cu, cp, cr, cu, cu, cu, cu, cu], axis=0)
    Ct = C.T
    out_ref[...] = (u * Ct[:, 0:1] + p * Ct[:, 1:2] + rel * Ct[:, 2:3])


def _edge_transform(comb4, et2, reltab, rtsq):
    grid = (E1 // BE,)
    nch = BE // 128
    return pl.pallas_call(
        _edge_body,
        grid=grid,
        in_specs=[
            pl.BlockSpec((nch, 2, 128, D), lambda i: (i, 0, 0, 0)),
            pl.BlockSpec((BE, 1), lambda i: (i, 0)),
            pl.BlockSpec((16, D), lambda i: (0, 0)),
            pl.BlockSpec((16, 1), lambda i: (0, 0)),
        ],
        out_specs=pl.BlockSpec((BE, D), lambda i: (i, 0)),
        out_shape=jax.ShapeDtypeStruct((E1, D), jnp.float32),
    )(comb4, et2, reltab, rtsq)


RF = 1000


def _fuse_body(e_ref, cf_ref, w1_ref, w2_ref, out_ref):
    e = e_ref[...]
    cf = cf_ref[...]
    g = jax.nn.sigmoid(
        jnp.dot(e, w1_ref[...], preferred_element_type=jnp.float32)
        + jnp.dot(cf, w2_ref[...], preferred_element_type=jnp.float32))
    out_ref[...] = g * e + (1.0 - g) * cf


def _fusion(ent_itm, cf, w1t, w2t):
    return pl.pallas_call(
        _fuse_body,
        grid=(N_ITM // RF,),
        in_specs=[
            pl.BlockSpec((RF, D), lambda i: (i, 0)),
            pl.BlockSpec((RF, D), lambda i: (i, 0)),
            pl.BlockSpec((D, D), lambda i: (0, 0)),
            pl.BlockSpec((D, D), lambda i: (0, 0)),
        ],
        out_specs=pl.BlockSpec((RF, D), lambda i: (i, 0)),
        out_shape=jax.ShapeDtypeStruct((N_ITM, D), jnp.float32),
    )(ent_itm, cf, w1t, w2t)


RB = 2528


def _final_body(s1_ref, c1_ref, s2_ref, c2_ref, o1_ref, o2_ref):
    s1 = s1_ref[0] + s1_ref[1]
    c1 = c1_ref[0, :, 0:1] + c1_ref[1, :, 0:1]
    o1_ref[...] = s1 / jnp.maximum(c1, 1.0)
    s2 = s2_ref[0] + s2_ref[1]
    c2 = c2_ref[0, :, 0:1] + c2_ref[1, :, 0:1]
    o2_ref[...] = s2 / jnp.maximum(c2, 1.0)


def _finalize(s1, c1, s2, c2):
    return pl.pallas_call(
        _final_body,
        grid=(NSEG // RB,),
        in_specs=[
            pl.BlockSpec((NC, RB, D), lambda i: (0, i, 0)),
            pl.BlockSpec((NC, RB, D), lambda i: (0, i, 0)),
            pl.BlockSpec((NC, RB, D), lambda i: (0, i, 0)),
            pl.BlockSpec((NC, RB, D), lambda i: (0, i, 0)),
        ],
        out_specs=(pl.BlockSpec((RB, D), lambda i: (i, 0)),
                   pl.BlockSpec((RB, D), lambda i: (i, 0))),
        out_shape=(jax.ShapeDtypeStruct((NSEG, D), jnp.float32),
                   jax.ShapeDtypeStruct((NSEG, D), jnp.float32)),
    )(s1, c1, s2, c2)


@jax.jit
def kernel(entity_embed, user_embed, relation_table, item_cf_embed, W1, W2,
           kg_src, kg_dst, edge_type, item_idx, user_idx):
    e_kg = kg_src.shape[0]
    e_int = item_idx.shape[0]

    pad1 = E1 - e_kg
    trash1 = TRASH + jnp.arange(pad1, dtype=jnp.int32) % (NSEG - TRASH)
    src_i = jnp.concatenate([kg_src, jnp.zeros((pad1,), jnp.int32)])
    dst_i = jnp.concatenate([kg_dst, jnp.zeros((pad1,), jnp.int32)])
    seg_i = jnp.concatenate([kg_src, trash1])
    et2 = jnp.concatenate([edge_type, jnp.zeros((pad1,), jnp.int32)])
    src_i = src_i.reshape(R1, 128)
    dst_i = dst_i.reshape(R1, 128)
    seg_i = seg_i.reshape(NW, G1, CH1, 128)
    et2 = et2.reshape(E1, 1)

    pad2 = E2 - 2 * e_int
    trash2 = TRASH + jnp.arange(pad2, dtype=jnp.int32) % (NSEG - TRASH)
    src2 = jnp.concatenate([item_idx, user_idx + N_ITM, trash2])
    dst2 = jnp.concatenate([user_idx + N_ITM, item_idx, trash2])
    src2 = src2.reshape(NW, G2, CH2, 128)
    dst2 = dst2.reshape(NW, G2, CH2, 128)

    reltab = jnp.concatenate(
        [relation_table, jnp.zeros((16 - relation_table.shape[0], D),
                                   jnp.float32)])
    rtsq = jnp.sum(reltab * reltab, axis=1, keepdims=True)
    ones_h = jnp.ones((128, D), jnp.float32)
    zer_s = jnp.zeros((NSEG, D), jnp.float32)

    comb = _kg_gather(entity_embed, src_i, dst_i)
    comb4 = comb.reshape(E1 // 128, 2, 128, D)

    c1, c2 = _counts(seg_i, dst2, ones_h, zer_s)

    fus = _fusion(entity_embed[:N_ITM], item_cf_embed, W1.T, W2.T)
    node = jnp.concatenate([fus, user_embed,
                            jnp.zeros((NSEG - N_ITM - N_USR, D),
                                      jnp.float32)])

    tan = _edge_transform(comb4, et2, reltab, rtsq)

    s1 = _kg_scatter(tan, seg_i, zer_s)

    s2 = _int_agg(node, src2, dst2, zer_s)

    o1, o2 = _finalize(s1, c1, s2, c2)

    out = o1[:N_ENT]
    u = o2[N_ITM:N_ITM + N_USR]
    i_cf = o2[:N_ITM]
    return (out, u, i_cf)


# speedup vs baseline: 1.6426x; 1.1758x over previous
import jax
import jax.numpy as jnp
from jax import lax
from jax.experimental import pallas as pl
from jax.experimental.pallas import tpu as pltpu
from jax.experimental.pallas import tpu_sc as plsc

EPS = 1e-5
MAX_NORM = 1.0 - 1e-3
D = 128
NC, NS = 2, 16
NW = NC * NS
N_ENT = 10000
N_ITM = 6000
N_USR = 4000

E1 = 327680
R1 = E1 // 128
K1 = R1 // NW
G1, CH1 = 10, 8
E2 = 425984
R2 = E2 // 128
K2 = R2 // NW
G2, CH2 = 13, 8
NSEG = 10112
TRASH = 10000
F = NSEG // NS


def _mk_mesh():
    return plsc.VectorSubcoreMesh(core_axis_name="c", subcore_axis_name="s",
                                  num_cores=NC, num_subcores=NS)


def _wid():
    return lax.axis_index("s") * NC + lax.axis_index("c")


K1H = K1 // 2


def _kg_gather_body(ent, src_i, dst_i, comb_o,
                    idx_s, idx_d, b0, b1, g0, g1, w0, w1):
    w = _wid()
    row0 = w * K1H
    pltpu.sync_copy(src_i.at[pl.ds(row0, K1H)], idx_s)
    pltpu.sync_copy(dst_i.at[pl.ds(row0, K1H)], idx_d)
    base = row0 * 256
    H = K1H // 2

    pltpu.make_async_copy(ent.at[idx_s.at[0]], b0.at[pl.ds(0, 128)], g0).start()
    pltpu.make_async_copy(ent.at[idx_d.at[0]], b0.at[pl.ds(128, 128)], g0).start()

    def grp(g, carry):
        j0 = 2 * g
        j1 = j0 + 1
        off0 = base + j0 * 256
        off1 = off0 + 256

        @pl.when(g > 0)
        def _():
            pltpu.make_async_copy(b1, comb_o.at[pl.ds(off1, 256)], w1).wait()

        pltpu.make_async_copy(ent.at[idx_s.at[j1]], b1.at[pl.ds(0, 128)], g1).start()
        pltpu.make_async_copy(ent.at[idx_d.at[j1]], b1.at[pl.ds(128, 128)], g1).start()

        pltpu.make_async_copy(ent.at[idx_s.at[j0]], b0.at[pl.ds(0, 128)], g0).wait()
        pltpu.make_async_copy(ent.at[idx_d.at[j0]], b0.at[pl.ds(128, 128)], g0).wait()
        pltpu.make_async_copy(b0, comb_o.at[pl.ds(off0, 256)], w0).start()

        @pl.when(g < H - 1)
        def _():
            pltpu.make_async_copy(b0, comb_o.at[pl.ds(off0, 256)], w0).wait()
            pltpu.make_async_copy(ent.at[idx_s.at[j0 + 2]], b0.at[pl.ds(0, 128)], g0).start()
            pltpu.make_async_copy(ent.at[idx_d.at[j0 + 2]], b0.at[pl.ds(128, 128)], g0).start()

        pltpu.make_async_copy(ent.at[idx_s.at[j1]], b1.at[pl.ds(0, 128)], g1).wait()
        pltpu.make_async_copy(ent.at[idx_d.at[j1]], b1.at[pl.ds(128, 128)], g1).wait()
        pltpu.make_async_copy(b1, comb_o.at[pl.ds(off1, 256)], w1).start()
        return carry

    lax.fori_loop(0, H, grp, 0)

    endo = base + (K1H - 2) * 256
    pltpu.make_async_copy(b0, comb_o.at[pl.ds(endo, 256)], w0).wait()
    pltpu.make_async_copy(b1, comb_o.at[pl.ds(endo + 256, 256)], w1).wait()


def _kg_gather(ent, src_i, dst_i):
    fn = pl.kernel(
        _kg_gather_body,
        out_type=jax.ShapeDtypeStruct((E1, D), jnp.float32),
        mesh=_mk_mesh(),
        name="sc_kg_gather",
        scratch_types=[
            pltpu.VMEM((K1H, 128), jnp.int32),
            pltpu.VMEM((K1H, 128), jnp.int32),
            pltpu.VMEM((256, D), jnp.float32),
            pltpu.VMEM((256, D), jnp.float32),
            pltpu.SemaphoreType.DMA,
            pltpu.SemaphoreType.DMA,
            pltpu.SemaphoreType.DMA,
            pltpu.SemaphoreType.DMA,
        ],
    )
    return fn(ent, src_i, dst_i)


def _kg_scatter_body(tan_a, tan_b, seg_i, zer_s, out_s,
                     acc, idx_v, b0, b1, r0, r1):
    c = lax.axis_index("c")
    s = lax.axis_index("s")
    w = s * NC + c
    pltpu.sync_copy(zer_s.at[pl.ds(s * F, F)], acc.at[pl.ds(s * F, F)])
    plsc.subcore_barrier()

    def run(tan, base):
        def group(g, carry):
            pltpu.sync_copy(seg_i.at[w, g], idx_v)
            goff = base + g * (CH1 * 128)
            pltpu.make_async_copy(tan.at[pl.ds(goff, 128)], b0, r0).start()

            def pair(p, carry2):
                j0 = 2 * p
                j1 = j0 + 1
                pltpu.make_async_copy(
                    tan.at[pl.ds(goff + j1 * 128, 128)], b1, r1).start()
                pltpu.make_async_copy(
                    tan.at[pl.ds(goff + j0 * 128, 128)], b0, r0).wait()
                pltpu.sync_copy(b0, acc.at[idx_v.at[j0]], add=True)

                @pl.when(p < CH1 // 2 - 1)
                def _():
                    pltpu.make_async_copy(
                        tan.at[pl.ds(goff + (j0 + 2) * 128, 128)], b0,
                        r0).start()

                pltpu.make_async_copy(
                    tan.at[pl.ds(goff + j1 * 128, 128)], b1, r1).wait()
                pltpu.sync_copy(b1, acc.at[idx_v.at[j1]], add=True)
                return carry2

            lax.fori_loop(0, CH1 // 2, pair, 0)
            return carry

        lax.fori_loop(0, G1, group, 0)

    @pl.when(w < NS)
    def _():
        run(tan_a, w * (K1 * 128))

    @pl.when(w >= NS)
    def _():
        run(tan_b, (w - NS) * (K1 * 128))

    plsc.subcore_barrier()
    pltpu.sync_copy(acc.at[pl.ds(s * F, F)], out_s.at[c].at[pl.ds(s * F, F)])


def _kg_scatter(tan_a, tan_b, seg_i, zer_s):
    fn = pl.kernel(
        _kg_scatter_body,
        out_type=jax.ShapeDtypeStruct((NC, NSEG, D), jnp.float32),
        mesh=_mk_mesh(),
        name="sc_kg_scatter",
        scratch_types=[
            pltpu.VMEM_SHARED((NSEG, D), jnp.float32),
            pltpu.VMEM((CH1, 128), jnp.int32),
            pltpu.VMEM((128, D), jnp.float32),
            pltpu.VMEM((128, D), jnp.float32),
            pltpu.SemaphoreType.DMA,
            pltpu.SemaphoreType.DMA,
        ],
    )
    return fn(tan_a, tan_b, seg_i, zer_s)


def _int_body(node, src_i, dst_i, zer_s, out_s,
              acc, idx_s, idx_d, b0, g0):
    c = lax.axis_index("c")
    s = lax.axis_index("s")
    w = s * NC + c
    pltpu.sync_copy(zer_s.at[pl.ds(s * F, F)], acc.at[pl.ds(s * F, F)])
    plsc.subcore_barrier()

    def group(g, carry):
        pltpu.sync_copy(src_i.at[w, g], idx_s)
        pltpu.sync_copy(dst_i.at[w, g], idx_d)

        def body(j, carry2):
            pltpu.make_async_copy(node.at[idx_s.at[j]], b0, g0).start()
            pltpu.make_async_copy(node.at[idx_s.at[j]], b0, g0).wait()
            pltpu.sync_copy(b0, acc.at[idx_d.at[j]], add=True)
            return carry2

        lax.fori_loop(0, CH2, body, 0)
        return carry

    lax.fori_loop(0, G2, group, 0)
    plsc.subcore_barrier()
    pltpu.sync_copy(acc.at[pl.ds(s * F, F)], out_s.at[c].at[pl.ds(s * F, F)])


def _int_agg(node, src_i, dst_i, zer_s):
    fn = pl.kernel(
        _int_body,
        out_type=jax.ShapeDtypeStruct((NC, NSEG, D), jnp.float32),
        mesh=_mk_mesh(),
        name="sc_int_agg",
        scratch_types=[
            pltpu.VMEM_SHARED((NSEG, D), jnp.float32),
            pltpu.VMEM((CH2, 128), jnp.int32),
            pltpu.VMEM((CH2, 128), jnp.int32),
            pltpu.VMEM((128, D), jnp.float32),
            pltpu.SemaphoreType.DMA,
        ],
    )
    return fn(node, src_i, dst_i, zer_s)


def _cnt_body(seg_i, dst_i, ones_h, zer_s, out_c1, out_c2,
              cnt, idx1, idx2, ones_v, sc):
    c = lax.axis_index("c")
    s = lax.axis_index("s")
    w = s * NC + c
    pltpu.sync_copy(zer_s.at[pl.ds(s * F, F)], cnt.at[pl.ds(s * F, F)])
    pltpu.sync_copy(ones_h, ones_v)
    plsc.subcore_barrier()

    def group1(g, carry):
        pltpu.sync_copy(seg_i.at[w, g], idx1)

        def fire(j, carry2):
            pltpu.make_async_copy(ones_v, cnt.at[idx1.at[j]], sc).start(
                add=True)
            return carry2

        lax.fori_loop(0, CH1, fire, 0)

        def drain(j, carry2):
            pltpu.make_async_copy(ones_v, cnt.at[idx1.at[j]], sc).wait()
            return carry2

        lax.fori_loop(0, CH1, drain, 0)
        return carry

    lax.fori_loop(0, G1, group1, 0)
    plsc.subcore_barrier()
    pltpu.sync_copy(cnt.at[pl.ds(s * F, F)], out_c1.at[c].at[pl.ds(s * F, F)])
    plsc.subcore_barrier()
    pltpu.sync_copy(zer_s.at[pl.ds(s * F, F)], cnt.at[pl.ds(s * F, F)])
    plsc.subcore_barrier()

    def group2(g, carry):
        pltpu.sync_copy(dst_i.at[w, g], idx2)

        def fire(j, carry2):
            pltpu.make_async_copy(ones_v, cnt.at[idx2.at[j]], sc).start(
                add=True)
            return carry2

        lax.fori_loop(0, CH2, fire, 0)

        def drain(j, carry2):
            pltpu.make_async_copy(ones_v, cnt.at[idx2.at[j]], sc).wait()
            return carry2

        lax.fori_loop(0, CH2, drain, 0)
        return carry

    lax.fori_loop(0, G2, group2, 0)
    plsc.subcore_barrier()
    pltpu.sync_copy(cnt.at[pl.ds(s * F, F)], out_c2.at[c].at[pl.ds(s * F, F)])


def _counts(seg_i, dst_i, ones_h, zer_s):
    fn = pl.kernel(
        _cnt_body,
        out_type=(jax.ShapeDtypeStruct((NC, NSEG, D), jnp.float32),
                  jax.ShapeDtypeStruct((NC, NSEG, D), jnp.float32)),
        mesh=_mk_mesh(),
        name="sc_counts",
        scratch_types=[
            pltpu.VMEM_SHARED((NSEG, D), jnp.float32),
            pltpu.VMEM((CH1, 128), jnp.int32),
            pltpu.VMEM((CH2, 128), jnp.int32),
            pltpu.VMEM((128, D), jnp.float32),
            pltpu.SemaphoreType.DMA,
        ],
    )
    return fn(seg_i, dst_i, ones_h, zer_s)


def _sq(x):
    return jnp.sum(x * x, axis=-1, keepdims=True)


BE = 1024


def _edge_body(comb_ref, et_ref, rt_ref, rtsq_ref, out_ref):
    u = comb_ref[:, 0].reshape(BE, D)
    p = comb_ref[:, 1].reshape(BE, D)
    et = et_ref[...]
    onehot = jnp.where(
        et + 2 == lax.broadcasted_iota(jnp.int32, (BE, 16), 1), 1.0, 0.0)
    rel = jnp.dot(onehot, rt_ref[...], preferred_element_type=jnp.float32)
    rr0 = jnp.dot(onehot, rtsq_ref[...], preferred_element_type=jnp.float32)

    uu0 = _sq(u)
    pp0 = _sq(p)
    up0 = jnp.sum(u * p, axis=-1, keepdims=True)
    ur0 = jnp.sum(u * rel, axis=-1, keepdims=True)
    pr0 = jnp.sum(p * rel, axis=-1, keepdims=True)
    S = jnp.concatenate([uu0, pp0, rr0, up0, ur0, pr0, uu0, uu0], axis=1)
    T = S.T
    uu = T[0:1]
    pp = T[1:2]
    rr = T[2:3]
    up = T[3:4]
    ur = T[4:5]
    pr = T[5:6]

    def qf(cu, cp, cr):
        return jnp.maximum(
            cu * cu * uu + cp * cp * pp + cr * cr * rr
            + 2.0 * (cu * cp * up + cu * cr * ur + cp * cr * pr), 0.0)

    n0 = jnp.maximum(jnp.sqrt(uu + 1e-15), EPS)
    sb0 = jnp.tanh(n0) / n0
    nb = jnp.sqrt(uu * sb0 * sb0 + 1e-15)
    fb = jnp.where(nb > MAX_NORM, MAX_NORM / nb, 1.0)
    sb = sb0 * fb
    bb = uu * sb * sb
    mb = jnp.maximum(1.0 - bb, EPS)

    def emap_coef(vv, uv):
        nv = jnp.maximum(jnp.sqrt(vv + 1e-15), EPS)
        sv = jnp.tanh(nv / mb) / nv
        y2 = vv * sv * sv
        xy = uv * sb * sv
        num_a = 1.0 + 2.0 * xy + y2
        num_b = 1.0 - bb
        rden = 1.0 / jnp.maximum(1.0 + 2.0 * xy + bb * y2, 1e-15)
        cb = num_a * rden * sb
        cv = num_b * rden * sv
        s2 = jnp.maximum(cb * cb * uu + 2.0 * cb * cv * uv + cv * cv * vv,
                         0.0)
        na = jnp.sqrt(s2 + 1e-15)
        fa = jnp.where(na > MAX_NORM, MAX_NORM / na, 1.0)
        return cb * fa, cv * fa, s2 * fa * fa

    au, ap, a2 = emap_coef(pp, up)
    bu, br, b2 = emap_coef(rr, ur)
    ab = au * bu * uu + au * br * ur + ap * bu * up + ap * br * pr
    a3 = 1.0 + 2.0 * ab + b2
    b3 = 1.0 - a2
    rd3 = 1.0 / jnp.maximum(1.0 + 2.0 * ab + a2 * b2, 1e-15)
    mu = a3 * rd3 * au + b3 * rd3 * bu
    mp = a3 * rd3 * ap
    mr = b3 * rd3 * br
    m2p = qf(mu, mp, mr)
    nm = jnp.sqrt(m2p + 1e-15)
    fm = jnp.where(nm > MAX_NORM, MAX_NORM / nm, 1.0)
    mu = mu * fm
    mp = mp * fm
    mr = mr * fm
    m2 = m2p * fm * fm
    bm = sb * (mu * uu + mp * up + mr * ur)
    a4 = 1.0 - 2.0 * bm + m2
    b4 = 1.0 - bb
    rd4 = 1.0 / jnp.maximum(1.0 - 2.0 * bm + bb * m2, 1e-15)
    su = -a4 * rd4 * sb + b4 * rd4 * mu
    sp = b4 * rd4 * mp
    sr = b4 * rd4 * mr
    s2s = qf(su, sp, sr)
    ns = jnp.clip(jnp.sqrt(s2s + 1e-15), EPS, 1.0 - 1e-5)
    atanh = 0.5 * jnp.log((1.0 + ns) / (1.0 - ns))
    scal = mb * atanh / ns
    cu = scal * su
    cp = scal * sp
    cr = scal * sr

    C = jnp.concatenate([cu, cp, cr, cu, cu, cu, cu, cu], axis=0)
    Ct = C.T
    out_ref[...] = (u * Ct[:, 0:1] + p * Ct[:, 1:2] + rel * Ct[:, 2:3])


def _edge_transform(comb4, et2, reltab, rtsq):
    grid = (E1 // 2 // BE,)
    nch = BE // 128
    return pl.pallas_call(
        _edge_body,
        grid=grid,
        in_specs=[
            pl.BlockSpec((nch, 2, 128, D), lambda i: (i, 0, 0, 0)),
            pl.BlockSpec((BE, 1), lambda i: (i, 0)),
            pl.BlockSpec((16, D), lambda i: (0, 0)),
            pl.BlockSpec((16, 1), lambda i: (0, 0)),
        ],
        out_specs=pl.BlockSpec((BE, D), lambda i: (i, 0)),
        out_shape=jax.ShapeDtypeStruct((E1 // 2, D), jnp.float32),
    )(comb4, et2, reltab, rtsq)


RF = 1000


def _fuse_body(e_ref, cf_ref, w1_ref, w2_ref, out_ref):
    e = e_ref[...]
    cf = cf_ref[...]
    g = jax.nn.sigmoid(
        jnp.dot(e, w1_ref[...], preferred_element_type=jnp.float32)
        + jnp.dot(cf, w2_ref[...], preferred_element_type=jnp.float32))
    out_ref[...] = g * e + (1.0 - g) * cf


def _fusion(ent_itm, cf, w1t, w2t):
    return pl.pallas_call(
        _fuse_body,
        grid=(N_ITM // RF,),
        in_specs=[
            pl.BlockSpec((RF, D), lambda i: (i, 0)),
            pl.BlockSpec((RF, D), lambda i: (i, 0)),
            pl.BlockSpec((D, D), lambda i: (0, 0)),
            pl.BlockSpec((D, D), lambda i: (0, 0)),
        ],
        out_specs=pl.BlockSpec((RF, D), lambda i: (i, 0)),
        out_shape=jax.ShapeDtypeStruct((N_ITM, D), jnp.float32),
    )(ent_itm, cf, w1t, w2t)


RB = 2528


def _final_body(s1_ref, c1_ref, s2_ref, c2_ref, o1_ref, o2_ref):
    s1 = s1_ref[0] + s1_ref[1]
    c1 = c1_ref[0, :, 0:1] + c1_ref[1, :, 0:1]
    o1_ref[...] = s1 / jnp.maximum(c1, 1.0)
    s2 = s2_ref[0] + s2_ref[1]
    c2 = c2_ref[0, :, 0:1] + c2_ref[1, :, 0:1]
    o2_ref[...] = s2 / jnp.maximum(c2, 1.0)


def _finalize(s1, c1, s2, c2):
    return pl.pallas_call(
        _final_body,
        grid=(NSEG // RB,),
        in_specs=[
            pl.BlockSpec((NC, RB, D), lambda i: (0, i, 0)),
            pl.BlockSpec((NC, RB, D), lambda i: (0, i, 0)),
            pl.BlockSpec((NC, RB, D), lambda i: (0, i, 0)),
            pl.BlockSpec((NC, RB, D), lambda i: (0, i, 0)),
        ],
        out_specs=(pl.BlockSpec((RB, D), lambda i: (i, 0)),
                   pl.BlockSpec((RB, D), lambda i: (i, 0))),
        out_shape=(jax.ShapeDtypeStruct((NSEG, D), jnp.float32),
                   jax.ShapeDtypeStruct((NSEG, D), jnp.float32)),
    )(s1, c1, s2, c2)


@jax.jit
def kernel(entity_embed, user_embed, relation_table, item_cf_embed, W1, W2,
           kg_src, kg_dst, edge_type, item_idx, user_idx):
    e_kg = kg_src.shape[0]
    e_int = item_idx.shape[0]

    pad1 = E1 - e_kg
    trash1 = TRASH + jnp.arange(pad1, dtype=jnp.int32) % (NSEG - TRASH)
    src_i = jnp.concatenate([kg_src, jnp.zeros((pad1,), jnp.int32)])
    dst_i = jnp.concatenate([kg_dst, jnp.zeros((pad1,), jnp.int32)])
    seg_i = jnp.concatenate([kg_src, trash1])
    et2 = jnp.concatenate([edge_type, jnp.zeros((pad1,), jnp.int32)])
    src_i = src_i.reshape(R1, 128)
    dst_i = dst_i.reshape(R1, 128)
    seg_i = seg_i.reshape(NW, G1, CH1, 128)
    et2 = et2.reshape(E1, 1)

    pad2 = E2 - 2 * e_int
    trash2 = TRASH + jnp.arange(pad2, dtype=jnp.int32) % (NSEG - TRASH)
    src2 = jnp.concatenate([item_idx, user_idx + N_ITM, trash2])
    dst2 = jnp.concatenate([user_idx + N_ITM, item_idx, trash2])
    src2 = src2.reshape(NW, G2, CH2, 128)
    dst2 = dst2.reshape(NW, G2, CH2, 128)

    reltab = jnp.concatenate(
        [relation_table, jnp.zeros((16 - relation_table.shape[0], D),
                                   jnp.float32)])
    rtsq = jnp.sum(reltab * reltab, axis=1, keepdims=True)
    ones_h = jnp.ones((128, D), jnp.float32)
    zer_s = jnp.zeros((NSEG, D), jnp.float32)

    comb_a = _kg_gather(entity_embed, src_i[:R1 // 2], dst_i[:R1 // 2])
    comb_b = _kg_gather(entity_embed, src_i[R1 // 2:], dst_i[R1 // 2:])
    comb4_a = comb_a.reshape(E1 // 256, 2, 128, D)
    comb4_b = comb_b.reshape(E1 // 256, 2, 128, D)

    c1, c2 = _counts(seg_i, dst2, ones_h, zer_s)

    fus = _fusion(entity_embed[:N_ITM], item_cf_embed, W1.T, W2.T)
    node = jnp.concatenate([fus, user_embed,
                            jnp.zeros((NSEG - N_ITM - N_USR, D),
                                      jnp.float32)])

    tan_a = _edge_transform(comb4_a, et2[:E1 // 2], reltab, rtsq)
    tan_b = _edge_transform(comb4_b, et2[E1 // 2:], reltab, rtsq)

    s1 = _kg_scatter(tan_a, tan_b, seg_i, zer_s)

    s2 = _int_agg(node, src2, dst2, zer_s)

    o1, o2 = _finalize(s1, c1, s2, c2)

    out = o1[:N_ENT]
    u = o2[N_ITM:N_ITM + N_USR]
    i_cf = o2[:N_ITM]
    return (out, u, i_cf)
